# baseline (speedup 1.0000x reference)
import functools

import jax
import jax.numpy as jnp
from jax import lax
from jax.experimental import pallas as pl
from jax.experimental.pallas import tpu as pltpu
from jax.experimental.pallas import tpu_sc as plsc

_DIM = 512
_L = 77
_XPOS = 5
_LPAD = 80
_ROWS_PER_W = 8
_N_WORKERS = _LPAD // _ROWS_PER_W
_NC = 2
_NS = 16
_BB = 128


def _sc_gather(idx_pad, table):
    @functools.partial(
        pl.kernel,
        out_type=jax.ShapeDtypeStruct((_LPAD, _DIM), jnp.float32),
        mesh=plsc.VectorSubcoreMesh(
            core_axis_name="c", subcore_axis_name="s",
            num_cores=_NC, num_subcores=_NS),
        scratch_types=[
            pltpu.VMEM((_ROWS_PER_W,), jnp.int32),
            pltpu.VMEM((_ROWS_PER_W, _DIM), jnp.float32),
            pltpu.SemaphoreType.DMA,
        ],
    )
    def gather(idx_hbm, table_hbm, out_hbm, idx_v, rows_v, sem):
        wid = lax.axis_index("s") * _NC + lax.axis_index("c")

        @pl.when(wid < _N_WORKERS)
        def _():
            base = wid * _ROWS_PER_W
            pltpu.sync_copy(idx_hbm.at[pl.ds(base, _ROWS_PER_W)], idx_v)
            pltpu.async_copy(table_hbm.at[idx_v], rows_v, sem).wait()
            pltpu.sync_copy(rows_v, out_hbm.at[pl.ds(base, _ROWS_PER_W)])

    return gather(idx_pad, table)


def _compose_body(tok_ref, emb_ref, s_ref, out_ref, tokb_ref):
    emb = emb_ref[...]
    s = s_ref[...]
    shape = out_ref.shape
    is_x = lax.broadcasted_iota(jnp.int32, shape, 1) == _XPOS
    out_ref[...] = jnp.where(
        is_x,
        jnp.broadcast_to(s[:, None, :], shape),
        jnp.broadcast_to(emb[None, :, :], shape),
    )
    tokb_ref[...] = jnp.broadcast_to(tok_ref[...], tokb_ref.shape)


def kernel(s_star, table, tokenized):
    bsz = s_star.shape[0]
    idx = tokenized.reshape(_L).astype(jnp.int32)
    idx_pad = jnp.pad(idx, (0, _LPAD - _L))
    emb = _sc_gather(idx_pad, table)[:_L]

    s_star = s_star.astype(jnp.float32)
    prompts, tok_b = pl.pallas_call(
        _compose_body,
        grid=(bsz // _BB,),
        in_specs=[
            pl.BlockSpec((1, _L), lambda i: (0, 0)),
            pl.BlockSpec((_L, _DIM), lambda i: (0, 0)),
            pl.BlockSpec((_BB, _DIM), lambda i: (i, 0)),
        ],
        out_specs=[
            pl.BlockSpec((_BB, _L, _DIM), lambda i: (i, 0, 0)),
            pl.BlockSpec((_BB, _L), lambda i: (i, 0)),
        ],
        out_shape=[
            jax.ShapeDtypeStruct((bsz, _L, _DIM), jnp.float32),
            jax.ShapeDtypeStruct((bsz, _L), jnp.int32),
        ],
    )(tokenized, emb, s_star)
    return prompts, tok_b
